# baseline (speedup 1.0000x reference)
import jax
import jax.numpy as jnp
from jax.experimental import pallas as pl
from jax.experimental.pallas import tpu as pltpu

_EPS = 1e-8


def _hgp_kernel(aux_ref, taux_ref, inc_ref, nf_ref, small_ref, itnf_ref, nfs_ref):
    inc = inc_ref[0]
    nf = nf_ref[0]
    a = aux_ref[0]
    t = taux_ref[0]
    H = inc.shape[0]
    N = inc.shape[1]

    e = a[0:1, :]
    topo = a[1:2, :]
    eta_raw = a[2:3, :]
    cosp = a[3:4, :]
    sinp = a[4:5, :]
    em = t[:, 4:5]

    ier = (inc * e) * topo
    s = ier.sum(axis=1, keepdims=True)
    denom = s + _EPS
    incn = ier / denom

    w = jnp.concatenate(
        [eta_raw * topo, cosp * topo, sinp * topo], axis=0
    )
    sums = jax.lax.dot_general(
        incn, w, (((1,), (1,)), ((), ())), preferred_element_type=jnp.float32
    )
    em_sum = jnp.dot(incn, em, preferred_element_type=jnp.float32)

    n_ke = jnp.log1p(jnp.maximum(jnp.maximum(s, 0.0), 0.0))
    n_eta = sums[:, 0:1]
    n_phi = jnp.arctan2(sums[:, 2:3], sums[:, 1:2])
    em_frac = em_sum

    mask_n = t[:, 0:1]
    zpad3 = jnp.zeros((H - N, 3), dtype=jnp.float32)
    zpad1 = jnp.zeros((H - N, 1), dtype=jnp.float32)
    ch3 = jnp.concatenate([t[:, 1:4] * mask_n, zpad3], axis=0)
    ch_mask = jnp.concatenate([mask_n, zpad1], axis=0)
    neut_keep = jnp.where(ch_mask > 0, 0.0, 1.0)

    small_ref[0] = jnp.concatenate(
        [
            ch3,
            n_ke * neut_keep,
            n_eta * neut_keep,
            n_phi * neut_keep,
            em_frac,
            ch_mask,
        ],
        axis=1,
    )

    itnf_ref[0] = jnp.dot(inc, nf, preferred_element_type=jnp.float32)
    nfs_ref[0] = nf.sum(axis=0, keepdims=True)


def kernel(pred_inc, node_feat, e_raw, is_topo, is_track, track_pt, track_eta,
           track_phi, topo_eta_raw, topo_phi, topo_em_frac):
    B, H, N = pred_inc.shape
    D = node_feat.shape[2]

    aux = jnp.stack(
        [e_raw, is_topo.astype(jnp.float32), topo_eta_raw,
         jnp.cos(topo_phi), jnp.sin(topo_phi)],
        axis=1,
    )
    zeros_bn = jnp.zeros_like(track_pt)
    taux = jnp.stack(
        [is_track.astype(jnp.float32), track_pt, track_eta, track_phi,
         topo_em_frac, zeros_bn, zeros_bn, zeros_bn], axis=2
    )

    small, itnf, nfs = pl.pallas_call(
        _hgp_kernel,
        grid=(B,),
        in_specs=[
            pl.BlockSpec((1, 5, N), lambda b: (b, 0, 0)),
            pl.BlockSpec((1, N, 8), lambda b: (b, 0, 0)),
            pl.BlockSpec((1, H, N), lambda b: (b, 0, 0)),
            pl.BlockSpec((1, N, D), lambda b: (b, 0, 0)),
        ],
        out_specs=[
            pl.BlockSpec((1, H, 8), lambda b: (b, 0, 0)),
            pl.BlockSpec((1, H, D), lambda b: (b, 0, 0)),
            pl.BlockSpec((1, 1, D), lambda b: (b, 0, 0)),
        ],
        out_shape=[
            jax.ShapeDtypeStruct((B, H, 8), jnp.float32),
            jax.ShapeDtypeStruct((B, H, D), jnp.float32),
            jax.ShapeDtypeStruct((B, 1, D), jnp.float32),
        ],
        compiler_params=pltpu.CompilerParams(
            dimension_semantics=("parallel",),
        ),
    )(aux, taux, pred_inc, node_feat)

    charged_proxy_kin = small[:, :, 0:3]
    neut_proxy_kin = small[:, :, 3:6]
    proxy_em_frac = small[:, :, 6]
    proxy_is_charged = small[:, :, 7] > 0
    node_feat_sum = nfs[:, 0, :]
    return (charged_proxy_kin, neut_proxy_kin, proxy_is_charged, proxy_em_frac,
            itnf, node_feat_sum)


# speedup vs baseline: 1.0248x; 1.0248x over previous
import jax
import jax.numpy as jnp
from jax.experimental import pallas as pl
from jax.experimental.pallas import tpu as pltpu

_EPS = 1e-8
_BB = 8


def _hgp_kernel(aux_ref, wn_ref, inc_ref, nf_ref, small_ref, itnf_ref, nfs_ref):
    H = inc_ref.shape[1]
    N = inc_ref.shape[2]
    for i in range(_BB):
        inc = inc_ref[i]
        nf = nf_ref[i]
        a = aux_ref[i]
        t = wn_ref[i]
        e = a[0:1, :]
        topo = a[1:2, :]

        ier = (inc * e) * topo
        s = ier.sum(axis=1, keepdims=True)
        denom = s + _EPS
        incn = ier / denom

        sums = jnp.dot(incn, t[:, 0:4],
                       preferred_element_type=jnp.float32)

        n_ke = jnp.log1p(jnp.maximum(jnp.maximum(s, 0.0), 0.0))
        n_eta = sums[:, 0:1]
        n_phi = jnp.arctan2(sums[:, 2:3], sums[:, 1:2])
        em_frac = sums[:, 3:4]

        mask_n = t[:, 4:5]
        zpad3 = jnp.zeros((H - N, 3), dtype=jnp.float32)
        zpad1 = jnp.zeros((H - N, 1), dtype=jnp.float32)
        ch3 = jnp.concatenate([t[:, 5:8] * mask_n, zpad3], axis=0)
        ch_mask = jnp.concatenate([mask_n, zpad1], axis=0)
        neut_keep = jnp.where(ch_mask > 0, 0.0, 1.0)

        small_ref[i] = jnp.concatenate(
            [
                ch3,
                n_ke * neut_keep,
                n_eta * neut_keep,
                n_phi * neut_keep,
                em_frac,
                ch_mask,
            ],
            axis=1,
        )

        itnf_ref[i] = jnp.dot(inc, nf, preferred_element_type=jnp.float32)
        nfs_ref[i] = nf.sum(axis=0, keepdims=True)


def kernel(pred_inc, node_feat, e_raw, is_topo, is_track, track_pt, track_eta,
           track_phi, topo_eta_raw, topo_phi, topo_em_frac):
    B, H, N = pred_inc.shape
    D = node_feat.shape[2]

    aux = jnp.stack([e_raw, is_topo.astype(jnp.float32)], axis=1)
    topo_f = is_topo.astype(jnp.float32)
    wn = jnp.stack(
        [topo_eta_raw * topo_f, jnp.cos(topo_phi) * topo_f,
         jnp.sin(topo_phi) * topo_f, topo_em_frac,
         is_track.astype(jnp.float32), track_pt, track_eta, track_phi], axis=2
    )

    small, itnf, nfs = pl.pallas_call(
        _hgp_kernel,
        grid=(B // _BB,),
        in_specs=[
            pl.BlockSpec((_BB, 2, N), lambda b: (b, 0, 0)),
            pl.BlockSpec((_BB, N, 8), lambda b: (b, 0, 0)),
            pl.BlockSpec((_BB, H, N), lambda b: (b, 0, 0)),
            pl.BlockSpec((_BB, N, D), lambda b: (b, 0, 0)),
        ],
        out_specs=[
            pl.BlockSpec((_BB, H, 8), lambda b: (b, 0, 0)),
            pl.BlockSpec((_BB, H, D), lambda b: (b, 0, 0)),
            pl.BlockSpec((_BB, 1, D), lambda b: (b, 0, 0)),
        ],
        out_shape=[
            jax.ShapeDtypeStruct((B, H, 8), jnp.float32),
            jax.ShapeDtypeStruct((B, H, D), jnp.float32),
            jax.ShapeDtypeStruct((B, 1, D), jnp.float32),
        ],
        compiler_params=pltpu.CompilerParams(
            dimension_semantics=("parallel",),
        ),
    )(aux, wn, pred_inc, node_feat)

    charged_proxy_kin = small[:, :, 0:3]
    neut_proxy_kin = small[:, :, 3:6]
    proxy_em_frac = small[:, :, 6]
    proxy_is_charged = small[:, :, 7] > 0
    node_feat_sum = nfs[:, 0, :]
    return (charged_proxy_kin, neut_proxy_kin, proxy_is_charged, proxy_em_frac,
            itnf, node_feat_sum)


# speedup vs baseline: 5.4111x; 5.2803x over previous
import jax
import jax.numpy as jnp
from jax.experimental import pallas as pl
from jax.experimental.pallas import tpu as pltpu

_EPS = 1e-8
_BB = 8


def _hgp_kernel(aux_ref, inc_ref, nf_ref, small_ref, itnf_ref, nfs_ref):
    H = inc_ref.shape[1]
    N = inc_ref.shape[2]
    ones_col = jnp.ones((N, 1), dtype=jnp.float32)
    for i in range(_BB):
        inc = inc_ref[i]
        nf = nf_ref[i]
        a = aux_ref[i]

        ier = inc * a[0:1, :]
        s = jnp.dot(ier, ones_col,
                    preferred_element_type=jnp.float32)
        denom = s + _EPS
        incn = ier / denom

        w4 = jnp.transpose(a[1:5, :], (1, 0))
        sums = jnp.dot(incn, w4,
                       preferred_element_type=jnp.float32)
        st = jnp.transpose(jnp.concatenate([sums, s], axis=1), (1, 0))

        ke = jnp.log1p(jnp.maximum(jnp.maximum(st[4:5, :], 0.0), 0.0))
        phi = jnp.arctan2(st[2:3, :], st[1:2, :])

        zpad = jnp.zeros((1, H - N), dtype=jnp.float32)
        mask_h = jnp.concatenate([a[5:6, :], zpad], axis=1)
        ch_pt = jnp.concatenate([a[6:7, :] * a[5:6, :], zpad], axis=1)
        ch_eta = jnp.concatenate([a[7:8, :] * a[5:6, :], zpad], axis=1)
        ch_phi = jnp.concatenate([a[8:9, :] * a[5:6, :], zpad], axis=1)
        keep = jnp.where(mask_h > 0, 0.0, 1.0)

        small_ref[i] = jnp.concatenate(
            [ch_pt, ch_eta, ch_phi,
             ke * keep, st[0:1, :] * keep, phi * keep,
             st[3:4, :], mask_h],
            axis=0,
        )

        itnf_ref[i] = jnp.dot(inc, nf, preferred_element_type=jnp.float32)
        nfs_ref[i] = nf.sum(axis=0, keepdims=True)


def kernel(pred_inc, node_feat, e_raw, is_topo, is_track, track_pt, track_eta,
           track_phi, topo_eta_raw, topo_phi, topo_em_frac):
    B, H, N = pred_inc.shape
    D = node_feat.shape[2]

    topo_f = is_topo.astype(jnp.float32)
    zeros_bn = jnp.zeros_like(e_raw)
    aux = jnp.stack(
        [e_raw * topo_f,
         topo_eta_raw * topo_f,
         jnp.cos(topo_phi) * topo_f,
         jnp.sin(topo_phi) * topo_f,
         topo_em_frac,
         is_track.astype(jnp.float32),
         track_pt,
         track_eta,
         track_phi,
         zeros_bn, zeros_bn, zeros_bn, zeros_bn, zeros_bn, zeros_bn, zeros_bn],
        axis=1,
    )

    small, itnf, nfs = pl.pallas_call(
        _hgp_kernel,
        grid=(B // _BB,),
        in_specs=[
            pl.BlockSpec((_BB, 16, N), lambda b: (b, 0, 0)),
            pl.BlockSpec((_BB, H, N), lambda b: (b, 0, 0)),
            pl.BlockSpec((_BB, N, D), lambda b: (b, 0, 0)),
        ],
        out_specs=[
            pl.BlockSpec((_BB, 8, H), lambda b: (b, 0, 0)),
            pl.BlockSpec((_BB, H, D), lambda b: (b, 0, 0)),
            pl.BlockSpec((_BB, 1, D), lambda b: (b, 0, 0)),
        ],
        out_shape=[
            jax.ShapeDtypeStruct((B, 8, H), jnp.float32),
            jax.ShapeDtypeStruct((B, H, D), jnp.float32),
            jax.ShapeDtypeStruct((B, 1, D), jnp.float32),
        ],
        compiler_params=pltpu.CompilerParams(
            dimension_semantics=("arbitrary",),
        ),
    )(aux, pred_inc, node_feat)

    charged_proxy_kin = jnp.transpose(small[:, 0:3, :], (0, 2, 1))
    neut_proxy_kin = jnp.transpose(small[:, 3:6, :], (0, 2, 1))
    proxy_em_frac = small[:, 6, :]
    proxy_is_charged = small[:, 7, :] > 0
    node_feat_sum = nfs[:, 0, :]
    return (charged_proxy_kin, neut_proxy_kin, proxy_is_charged, proxy_em_frac,
            itnf, node_feat_sum)


# speedup vs baseline: 6.6566x; 1.2302x over previous
import jax
import jax.numpy as jnp
from jax.experimental import pallas as pl
from jax.experimental.pallas import tpu as pltpu

_EPS = 1e-8
_BB = 8


def _hgp_kernel(aux_ref, inc_ref, nf_ref, small_ref, itnf_ref, nfs_ref):
    H = inc_ref.shape[1]
    N = inc_ref.shape[2]
    ones_col = jnp.ones((N, 1), dtype=jnp.float32)
    for i in range(_BB):
        inc = inc_ref[i]
        nf = nf_ref[i]
        a = aux_ref[i]

        ier = inc * a[0:1, :]
        s = ier.sum(axis=1, keepdims=True)
        denom = s + _EPS
        incn = ier / denom

        w4 = jnp.transpose(a[1:5, :], (1, 0))
        sums = jnp.dot(incn, w4,
                       preferred_element_type=jnp.float32)
        st = jnp.transpose(jnp.concatenate([sums, s], axis=1), (1, 0))

        ke = jnp.log1p(jnp.maximum(jnp.maximum(st[4:5, :], 0.0), 0.0))
        phi = jnp.arctan2(st[2:3, :], st[1:2, :])

        zpad = jnp.zeros((1, H - N), dtype=jnp.float32)
        mask_h = jnp.concatenate([a[5:6, :], zpad], axis=1)
        ch_pt = jnp.concatenate([a[6:7, :] * a[5:6, :], zpad], axis=1)
        ch_eta = jnp.concatenate([a[7:8, :] * a[5:6, :], zpad], axis=1)
        ch_phi = jnp.concatenate([a[8:9, :] * a[5:6, :], zpad], axis=1)
        keep = jnp.where(mask_h > 0, 0.0, 1.0)

        small_ref[i] = jnp.concatenate(
            [ch_pt, ch_eta, ch_phi,
             ke * keep, st[0:1, :] * keep, phi * keep,
             st[3:4, :], mask_h],
            axis=0,
        )

        itnf_ref[i] = jnp.dot(inc, nf, preferred_element_type=jnp.float32)
        nfs_ref[i] = nf.sum(axis=0, keepdims=True)


def kernel(pred_inc, node_feat, e_raw, is_topo, is_track, track_pt, track_eta,
           track_phi, topo_eta_raw, topo_phi, topo_em_frac):
    B, H, N = pred_inc.shape
    D = node_feat.shape[2]

    topo_f = is_topo.astype(jnp.float32)
    zeros_bn = jnp.zeros_like(e_raw)
    aux = jnp.stack(
        [e_raw * topo_f,
         topo_eta_raw * topo_f,
         jnp.cos(topo_phi) * topo_f,
         jnp.sin(topo_phi) * topo_f,
         topo_em_frac,
         is_track.astype(jnp.float32),
         track_pt,
         track_eta,
         track_phi,
         zeros_bn, zeros_bn, zeros_bn, zeros_bn, zeros_bn, zeros_bn, zeros_bn],
        axis=1,
    )

    small, itnf, nfs = pl.pallas_call(
        _hgp_kernel,
        grid=(B // _BB,),
        in_specs=[
            pl.BlockSpec((_BB, 16, N), lambda b: (b, 0, 0)),
            pl.BlockSpec((_BB, H, N), lambda b: (b, 0, 0)),
            pl.BlockSpec((_BB, N, D), lambda b: (b, 0, 0)),
        ],
        out_specs=[
            pl.BlockSpec((_BB, 8, H), lambda b: (b, 0, 0)),
            pl.BlockSpec((_BB, H, D), lambda b: (b, 0, 0)),
            pl.BlockSpec((_BB, 1, D), lambda b: (b, 0, 0)),
        ],
        out_shape=[
            jax.ShapeDtypeStruct((B, 8, H), jnp.float32),
            jax.ShapeDtypeStruct((B, H, D), jnp.float32),
            jax.ShapeDtypeStruct((B, 1, D), jnp.float32),
        ],
        compiler_params=pltpu.CompilerParams(
            dimension_semantics=("arbitrary",),
        ),
    )(aux, pred_inc, node_feat)

    charged_proxy_kin = jnp.transpose(small[:, 0:3, :], (0, 2, 1))
    neut_proxy_kin = jnp.transpose(small[:, 3:6, :], (0, 2, 1))
    proxy_em_frac = small[:, 6, :]
    proxy_is_charged = small[:, 7, :] > 0
    node_feat_sum = nfs[:, 0, :]
    return (charged_proxy_kin, neut_proxy_kin, proxy_is_charged, proxy_em_frac,
            itnf, node_feat_sum)
